# baseline (speedup 1.0000x reference)
import jax
import jax.numpy as jnp
from jax import lax
from jax.experimental import pallas as pl
from jax.experimental.pallas import tpu as pltpu
from jax.experimental.pallas import tpu_sc as plsc

_NC = 2
_NS = 16
_K = 128
_G = 16


def _make_segsum(n_pad, h, e_pad, with_deg):
    nw = _NC * _NS
    nch = e_pad // (nw * _K)
    ngrp = nch // _G
    rpt = n_pad // _NS
    mesh = plsc.VectorSubcoreMesh(core_axis_name="c", subcore_axis_name="s")

    out_type = [jax.ShapeDtypeStruct((_NC, n_pad, h), jnp.float32)]
    if with_deg:
        out_type.append(jax.ShapeDtypeStruct((_NC * n_pad,), jnp.float32))

    scratch = [
        pltpu.VMEM((_G, _K), jnp.int32),
        pltpu.VMEM((_G, _K), jnp.int32),
        pltpu.VMEM((_K, h), jnp.float32),
        pltpu.VMEM((_K, h), jnp.float32),
        pltpu.VMEM_SHARED((n_pad, h), jnp.float32),
        pltpu.SemaphoreType.DMA,
        pltpu.SemaphoreType.DMA,
    ]
    if with_deg:
        scratch += [
            pltpu.VMEM((_K,), jnp.float32),
            pltpu.VMEM_SHARED((n_pad,), jnp.float32),
        ]

    def body(y, src2d, dst2d, zrows, *refs):
        if with_deg:
            (zvec, aggout, degout, sidx, didx, rows, rows2, agg_sp,
             gsem, gsem2, ones_v, deg_sp) = refs
        else:
            aggout, sidx, didx, rows, rows2, agg_sp, gsem, gsem2 = refs
        c = lax.axis_index("c")
        s = lax.axis_index("s")
        wid = c * _NS + s

        r0 = s * rpt
        pltpu.sync_copy(zrows, agg_sp.at[pl.ds(r0, rpt), :])
        if with_deg:
            pltpu.sync_copy(zvec, deg_sp.at[pl.ds(r0, rpt)])
            one16 = jnp.ones((16,), jnp.float32)
            def fo(j, _):
                ones_v[pl.ds(j * 16, 16)] = one16
                return 0
            lax.fori_loop(0, _K // 16, fo, 0)
        plsc.subcore_barrier()

        def grp(g, _):
            pltpu.sync_copy(src2d.at[wid, pl.ds(g * _G, _G)], sidx)
            pltpu.sync_copy(dst2d.at[wid, pl.ds(g * _G, _G)], didx)

            def pair(j, __):
                c0 = 2 * j
                ha = pltpu.async_copy(y.at[sidx.at[c0]], rows, gsem)
                hb = pltpu.async_copy(y.at[sidx.at[c0 + 1]], rows2, gsem2)
                ha.wait()
                pltpu.sync_copy(rows, agg_sp.at[didx.at[c0]], add=True)
                if with_deg:
                    pltpu.sync_copy(ones_v, deg_sp.at[didx.at[c0]], add=True)
                hb.wait()
                pltpu.sync_copy(rows2, agg_sp.at[didx.at[c0 + 1]], add=True)
                if with_deg:
                    pltpu.sync_copy(ones_v, deg_sp.at[didx.at[c0 + 1]], add=True)
                return 0
            lax.fori_loop(0, _G // 2, pair, 0)
            return 0
        lax.fori_loop(0, ngrp, grp, 0)
        plsc.subcore_barrier()

        pltpu.sync_copy(agg_sp.at[pl.ds(r0, rpt), :], aggout.at[c, pl.ds(r0, rpt), :])
        if with_deg:
            pltpu.sync_copy(deg_sp.at[pl.ds(r0, rpt)],
                            degout.at[pl.ds(c * n_pad + r0, rpt)])

    return pl.kernel(body, out_type=out_type, mesh=mesh, scratch_types=scratch)


_RB = 2000


def _matmul(x, w):
    n, d = x.shape
    h = w.shape[1]

    def tc_body(x_ref, w_ref, y_ref):
        y_ref[...] = jnp.dot(x_ref[...], w_ref[...],
                             preferred_element_type=jnp.float32)

    return pl.pallas_call(
        tc_body,
        grid=(n // _RB,),
        in_specs=[
            pl.BlockSpec((_RB, d), lambda i: (i, 0)),
            pl.BlockSpec((d, h), lambda i: (0, 0)),
        ],
        out_specs=pl.BlockSpec((_RB, h), lambda i: (i, 0)),
        out_shape=jax.ShapeDtypeStruct((n, h), jnp.float32),
    )(x, w)


def _combine2(p, deg2, r, b, wl):
    h = r.shape[1]
    n = r.shape[0]
    hn_y = wl.shape[1]

    def tc_body(p_ref, d_ref, r_ref, b_ref, wl_ref, y_ref, h_ref):
        agg = p_ref[0] + p_ref[1]
        inv = 1.0 / jnp.maximum(d_ref[0] + d_ref[1], 1.0)
        hh = jnp.maximum(agg * inv + r_ref[...] + b_ref[...], 0.0)
        y_ref[...] = jnp.dot(hh, wl_ref[...], preferred_element_type=jnp.float32)
        h_ref[...] = hh

    return pl.pallas_call(
        tc_body,
        grid=(n // _RB,),
        in_specs=[
            pl.BlockSpec((_NC, _RB, h), lambda i: (0, i, 0)),
            pl.BlockSpec((_NC, _RB, 1), lambda i: (0, i, 0)),
            pl.BlockSpec((_RB, h), lambda i: (i, 0)),
            pl.BlockSpec((1, h), lambda i: (0, 0)),
            pl.BlockSpec((h, hn_y), lambda i: (0, 0)),
        ],
        out_specs=[
            pl.BlockSpec((_RB, hn_y), lambda i: (i, 0)),
            pl.BlockSpec((_RB, h), lambda i: (i, 0)),
        ],
        out_shape=[
            jax.ShapeDtypeStruct((n, hn_y), jnp.float32),
            jax.ShapeDtypeStruct((n, h), jnp.float32),
        ],
    )(p, deg2, r, b, wl)


def _final(p, deg2, r, b):
    cdim = r.shape[1]
    n = r.shape[0]
    pw = p.shape[2]

    def tc_body(p_ref, d_ref, r_ref, b_ref, o_ref):
        agg = p_ref[0, :, :cdim] + p_ref[1, :, :cdim]
        inv = 1.0 / jnp.maximum(d_ref[0] + d_ref[1], 1.0)
        o_ref[...] = jnp.tanh(agg * inv + r_ref[...] + b_ref[...])

    return pl.pallas_call(
        tc_body,
        grid=(n // _RB,),
        in_specs=[
            pl.BlockSpec((_NC, _RB, pw), lambda i: (0, i, 0)),
            pl.BlockSpec((_NC, _RB, 1), lambda i: (0, i, 0)),
            pl.BlockSpec((_RB, cdim), lambda i: (i, 0)),
            pl.BlockSpec((1, cdim), lambda i: (0, 0)),
        ],
        out_specs=pl.BlockSpec((_RB, cdim), lambda i: (i, 0)),
        out_shape=jax.ShapeDtypeStruct((n, cdim), jnp.float32),
    )(p, deg2, r, b)


def kernel(x, edge_index, W_l0, W_r0, b0, W_l1, W_r1, b1, W_l2, W_r2, b2):
    n, _ = x.shape
    e = edge_index.shape[1]
    h = W_l0.shape[1]
    cdim = W_l2.shape[1]
    n_pad = -(-(n + 1) // (_NS * 16)) * (_NS * 16)
    nw = _NC * _NS

    et = e // nw
    nch = -(-et // (_K * _G)) * _G
    ppt = nch * _K - et
    e_pad = nw * nch * _K
    src_t = edge_index[0].reshape(nw, et)
    dst_t = edge_index[1].reshape(nw, et)
    if ppt:
        spare = n_pad - n
        pad_dst = n + (jnp.arange(ppt, dtype=jnp.int32) % spare)
        pad_src = (jnp.arange(ppt, dtype=jnp.int32) * 97) % n
        src_t = jnp.concatenate(
            [src_t, jnp.broadcast_to(pad_src, (nw, ppt))], axis=1)
        dst_t = jnp.concatenate(
            [dst_t, jnp.broadcast_to(pad_dst, (nw, ppt))], axis=1)
    src2d = src_t.reshape(nw, nch, _K)
    dst2d = dst_t.reshape(nw, nch, _K)

    seg_deg = _make_segsum(n_pad, h, e_pad, True)
    seg_h = _make_segsum(n_pad, h, e_pad, False)

    wl2p = jnp.concatenate([W_l2, jnp.zeros((h, h - cdim), jnp.float32)], axis=1)

    rpt = n_pad // _NS
    zrows = jnp.zeros((rpt, h), jnp.float32)
    zvec = jnp.zeros((rpt,), jnp.float32)

    y0 = _matmul(x, W_l0)
    p0, degp = seg_deg(y0, src2d, dst2d, zrows, zvec)
    r0 = _matmul(x, W_r0)
    deg2 = degp.reshape(_NC, n_pad, 1)
    y1, h1 = _combine2(p0, deg2, r0, b0.reshape(1, h), W_l1)
    p1 = seg_h(y1, src2d, dst2d, zrows)
    r1 = _matmul(h1, W_r1)
    if isinstance(p1, (list, tuple)):
        p1 = p1[0]
    y2, h2 = _combine2(p1, deg2, r1, b1.reshape(1, h), wl2p)
    p2 = seg_h(y2, src2d, dst2d, zrows)
    r2 = _matmul(h2, W_r2)
    if isinstance(p2, (list, tuple)):
        p2 = p2[0]
    return _final(p2, deg2, r2, b2.reshape(1, cdim))


# speedup vs baseline: 1.0361x; 1.0361x over previous
import jax
import jax.numpy as jnp
from jax import lax
from jax.experimental import pallas as pl
from jax.experimental.pallas import tpu as pltpu
from jax.experimental.pallas import tpu_sc as plsc

_NC = 2
_NS = 16
_K = 128
_G = 16


def _make_segsum(n_pad, h, e_pad, with_deg):
    nw = _NC * _NS
    nch = e_pad // (nw * _K)
    ngrp = nch // _G
    rpt = n_pad // _NS
    mesh = plsc.VectorSubcoreMesh(core_axis_name="c", subcore_axis_name="s")

    out_type = [jax.ShapeDtypeStruct((_NC, n_pad, h), jnp.float32)]
    if with_deg:
        out_type.append(jax.ShapeDtypeStruct((_NC * n_pad,), jnp.float32))

    scratch = [
        pltpu.VMEM((_G, _K), jnp.int32),
        pltpu.VMEM((_G, _K), jnp.int32),
        pltpu.VMEM((_K, h), jnp.float32),
        pltpu.VMEM((_K, h), jnp.float32),
        pltpu.VMEM_SHARED((n_pad, h), jnp.float32),
        pltpu.SemaphoreType.DMA,
        pltpu.SemaphoreType.DMA,
    ]
    if with_deg:
        scratch += [
            pltpu.VMEM((_K,), jnp.float32),
            pltpu.VMEM_SHARED((n_pad,), jnp.float32),
        ]

    def body(y, src2d, dst2d, *refs):
        if with_deg:
            (aggout, degout, sidx, didx, rows, rows2, agg_sp,
             gsem, gsem2, ones_v, deg_sp) = refs
        else:
            aggout, sidx, didx, rows, rows2, agg_sp, gsem, gsem2 = refs
        c = lax.axis_index("c")
        s = lax.axis_index("s")
        wid = c * _NS + s
        zero16 = jnp.zeros((16,), jnp.float32)

        def zr(i, _):
            def zc(j, __):
                rows[i, pl.ds(j * 16, 16)] = zero16
                return 0
            return lax.fori_loop(0, h // 16, zc, 0)
        lax.fori_loop(0, _K, zr, 0)
        r0 = s * rpt
        ztail = rpt - (rpt // _K) * _K
        for kk in range(rpt // _K):
            pltpu.sync_copy(rows, agg_sp.at[pl.ds(r0 + kk * _K, _K), :])
        if ztail:
            pltpu.sync_copy(rows.at[pl.ds(0, ztail), :],
                            agg_sp.at[pl.ds(r0 + (rpt // _K) * _K, ztail), :])
        if with_deg:
            def zo(j, _):
                ones_v[pl.ds(j * 16, 16)] = zero16
                return 0
            lax.fori_loop(0, _K // 16, zo, 0)
            for kk in range(rpt // _K):
                pltpu.sync_copy(ones_v, deg_sp.at[pl.ds(r0 + kk * _K, _K)])
            if ztail:
                pltpu.sync_copy(ones_v.at[pl.ds(0, ztail)],
                                deg_sp.at[pl.ds(r0 + (rpt // _K) * _K, ztail)])
            one16 = jnp.ones((16,), jnp.float32)
            def fo(j, _):
                ones_v[pl.ds(j * 16, 16)] = one16
                return 0
            lax.fori_loop(0, _K // 16, fo, 0)
        plsc.subcore_barrier()

        def grp(g, _):
            pltpu.sync_copy(src2d.at[wid, pl.ds(g * _G, _G)], sidx)
            pltpu.sync_copy(dst2d.at[wid, pl.ds(g * _G, _G)], didx)

            def pair(j, __):
                c0 = 2 * j
                ha = pltpu.async_copy(y.at[sidx.at[c0]], rows, gsem)
                hb = pltpu.async_copy(y.at[sidx.at[c0 + 1]], rows2, gsem2)
                ha.wait()
                pltpu.sync_copy(rows, agg_sp.at[didx.at[c0]], add=True)
                if with_deg:
                    pltpu.sync_copy(ones_v, deg_sp.at[didx.at[c0]], add=True)
                hb.wait()
                pltpu.sync_copy(rows2, agg_sp.at[didx.at[c0 + 1]], add=True)
                if with_deg:
                    pltpu.sync_copy(ones_v, deg_sp.at[didx.at[c0 + 1]], add=True)
                return 0
            lax.fori_loop(0, _G // 2, pair, 0)
            return 0
        lax.fori_loop(0, ngrp, grp, 0)
        plsc.subcore_barrier()

        pltpu.sync_copy(agg_sp.at[pl.ds(r0, rpt), :], aggout.at[c, pl.ds(r0, rpt), :])
        if with_deg:
            pltpu.sync_copy(deg_sp.at[pl.ds(r0, rpt)],
                            degout.at[pl.ds(c * n_pad + r0, rpt)])

    return pl.kernel(body, out_type=out_type, mesh=mesh, scratch_types=scratch)


_RB = 2000


def _matmul(x, w):
    n, d = x.shape
    h = w.shape[1]

    def tc_body(x_ref, w_ref, y_ref):
        y_ref[...] = jnp.dot(x_ref[...], w_ref[...],
                             preferred_element_type=jnp.float32)

    return pl.pallas_call(
        tc_body,
        grid=(n // _RB,),
        in_specs=[
            pl.BlockSpec((_RB, d), lambda i: (i, 0)),
            pl.BlockSpec((d, h), lambda i: (0, 0)),
        ],
        out_specs=pl.BlockSpec((_RB, h), lambda i: (i, 0)),
        out_shape=jax.ShapeDtypeStruct((n, h), jnp.float32),
    )(x, w)


def _combine2(p, deg2, r, b, wl):
    h = r.shape[1]
    n = r.shape[0]
    hn_y = wl.shape[1]

    def tc_body(p_ref, d_ref, r_ref, b_ref, wl_ref, y_ref, h_ref):
        agg = p_ref[0] + p_ref[1]
        inv = 1.0 / jnp.maximum(d_ref[0] + d_ref[1], 1.0)
        hh = jnp.maximum(agg * inv + r_ref[...] + b_ref[...], 0.0)
        y_ref[...] = jnp.dot(hh, wl_ref[...], preferred_element_type=jnp.float32)
        h_ref[...] = hh

    return pl.pallas_call(
        tc_body,
        grid=(n // _RB,),
        in_specs=[
            pl.BlockSpec((_NC, _RB, h), lambda i: (0, i, 0)),
            pl.BlockSpec((_NC, _RB, 1), lambda i: (0, i, 0)),
            pl.BlockSpec((_RB, h), lambda i: (i, 0)),
            pl.BlockSpec((1, h), lambda i: (0, 0)),
            pl.BlockSpec((h, hn_y), lambda i: (0, 0)),
        ],
        out_specs=[
            pl.BlockSpec((_RB, hn_y), lambda i: (i, 0)),
            pl.BlockSpec((_RB, h), lambda i: (i, 0)),
        ],
        out_shape=[
            jax.ShapeDtypeStruct((n, hn_y), jnp.float32),
            jax.ShapeDtypeStruct((n, h), jnp.float32),
        ],
    )(p, deg2, r, b, wl)


def _final(p, deg2, r, b):
    cdim = r.shape[1]
    n = r.shape[0]
    pw = p.shape[2]

    def tc_body(p_ref, d_ref, r_ref, b_ref, o_ref):
        agg = p_ref[0, :, :cdim] + p_ref[1, :, :cdim]
        inv = 1.0 / jnp.maximum(d_ref[0] + d_ref[1], 1.0)
        o_ref[...] = jnp.tanh(agg * inv + r_ref[...] + b_ref[...])

    return pl.pallas_call(
        tc_body,
        grid=(n // _RB,),
        in_specs=[
            pl.BlockSpec((_NC, _RB, pw), lambda i: (0, i, 0)),
            pl.BlockSpec((_NC, _RB, 1), lambda i: (0, i, 0)),
            pl.BlockSpec((_RB, cdim), lambda i: (i, 0)),
            pl.BlockSpec((1, cdim), lambda i: (0, 0)),
        ],
        out_specs=pl.BlockSpec((_RB, cdim), lambda i: (i, 0)),
        out_shape=jax.ShapeDtypeStruct((n, cdim), jnp.float32),
    )(p, deg2, r, b)


def kernel(x, edge_index, W_l0, W_r0, b0, W_l1, W_r1, b1, W_l2, W_r2, b2):
    n, _ = x.shape
    e = edge_index.shape[1]
    h = W_l0.shape[1]
    cdim = W_l2.shape[1]
    n_pad = -(-(n + 1) // (_NS * 16)) * (_NS * 16)
    nw = _NC * _NS

    et = e // nw
    nch = -(-et // (_K * _G)) * _G
    ppt = nch * _K - et
    e_pad = nw * nch * _K
    src_t = edge_index[0].reshape(nw, et)
    dst_t = edge_index[1].reshape(nw, et)
    if ppt:
        spare = n_pad - n
        pad_dst = n + (jnp.arange(ppt, dtype=jnp.int32) % spare)
        pad_src = (jnp.arange(ppt, dtype=jnp.int32) * 97) % n
        src_t = jnp.concatenate(
            [src_t, jnp.broadcast_to(pad_src, (nw, ppt))], axis=1)
        dst_t = jnp.concatenate(
            [dst_t, jnp.broadcast_to(pad_dst, (nw, ppt))], axis=1)
    src2d = src_t.reshape(nw, nch, _K)
    dst2d = dst_t.reshape(nw, nch, _K)

    seg_deg = _make_segsum(n_pad, h, e_pad, True)
    seg_h = _make_segsum(n_pad, h, e_pad, False)

    wl2p = jnp.concatenate([W_l2, jnp.zeros((h, h - cdim), jnp.float32)], axis=1)

    y0 = _matmul(x, W_l0)
    p0, degp = seg_deg(y0, src2d, dst2d)
    r0 = _matmul(x, W_r0)
    deg2 = degp.reshape(_NC, n_pad, 1)
    y1, h1 = _combine2(p0, deg2, r0, b0.reshape(1, h), W_l1)
    p1 = seg_h(y1, src2d, dst2d)
    r1 = _matmul(h1, W_r1)
    if isinstance(p1, (list, tuple)):
        p1 = p1[0]
    y2, h2 = _combine2(p1, deg2, r1, b1.reshape(1, h), wl2p)
    p2 = seg_h(y2, src2d, dst2d)
    r2 = _matmul(h2, W_r2)
    if isinstance(p2, (list, tuple)):
        p2 = p2[0]
    return _final(p2, deg2, r2, b2.reshape(1, cdim))
